# baseline (device time: 14169 ns/iter reference)
import jax
import jax.numpy as jnp
from jax import lax
from jax.experimental import pallas as pl
from jax.experimental.pallas import tpu as pltpu

N_DEV = 16
EPS = 1e-5


def kernel(x, t_emb, W_scale, W_shift):
    b, s, c = x.shape
    c_global = c * N_DEV

    def body(x_ref, t_ref, ws_ref, wsh_ref, out_ref,
             statbuf, send_sems, recv_sems):
        my = lax.axis_index("i")

        barrier = pltpu.get_barrier_semaphore()
        for d in range(1, N_DEV):
            peer = lax.rem(my + d, N_DEV)
            pl.semaphore_signal(
                barrier, inc=1,
                device_id=(peer,), device_id_type=pl.DeviceIdType.MESH,
            )
        pl.semaphore_wait(barrier, N_DEV - 1)

        xv = x_ref[...]
        s1 = jnp.sum(xv, axis=-1)
        s2 = jnp.sum(xv * xv, axis=-1)
        statbuf[pl.ds(my, 1)] = jnp.stack([s1, s2], axis=0)[None]

        sends = []
        for d in range(1, N_DEV):
            dst = lax.rem(my + d, N_DEV)
            rdma = pltpu.make_async_remote_copy(
                src_ref=statbuf.at[my],
                dst_ref=statbuf.at[my],
                send_sem=send_sems.at[d - 1],
                recv_sem=recv_sems.at[my],
                device_id=(dst,),
                device_id_type=pl.DeviceIdType.MESH,
            )
            rdma.start()
            sends.append(rdma)

        scale = jnp.dot(t_ref[...], ws_ref[...],
                        preferred_element_type=jnp.float32)
        shift = jnp.dot(t_ref[...], wsh_ref[...],
                        preferred_element_type=jnp.float32)

        for d in range(1, N_DEV):
            src = lax.rem(my + d, N_DEV)
            recv = pltpu.make_async_remote_copy(
                src_ref=statbuf.at[my],
                dst_ref=statbuf.at[src],
                send_sem=send_sems.at[d - 1],
                recv_sem=recv_sems.at[src],
                device_id=(my,),
                device_id_type=pl.DeviceIdType.MESH,
            )
            recv.wait_recv()
        for rdma in sends:
            rdma.wait_send()

        tot = jnp.sum(statbuf[...], axis=0)
        mean = tot[0] / c_global
        var = tot[1] / c_global - mean * mean
        inv = lax.rsqrt(var + EPS)
        h = (xv - mean[..., None]) * inv[..., None]
        out_ref[...] = h * (1.0 + scale[:, None, :]) + shift[:, None, :]

    return pl.pallas_call(
        body,
        out_shape=jax.ShapeDtypeStruct((b, s, c), jnp.float32),
        in_specs=[
            pl.BlockSpec(memory_space=pltpu.VMEM),
            pl.BlockSpec(memory_space=pltpu.VMEM),
            pl.BlockSpec(memory_space=pltpu.VMEM),
            pl.BlockSpec(memory_space=pltpu.VMEM),
        ],
        out_specs=pl.BlockSpec(memory_space=pltpu.VMEM),
        scratch_shapes=[
            pltpu.VMEM((N_DEV, 2, b, s), jnp.float32),
            pltpu.SemaphoreType.DMA((N_DEV - 1,)),
            pltpu.SemaphoreType.DMA((N_DEV,)),
        ],
        compiler_params=pltpu.CompilerParams(collective_id=0),
    )(x, t_emb, W_scale, W_shift)


# device time: 13473 ns/iter; 1.0517x vs baseline; 1.0517x over previous
import jax
import jax.numpy as jnp
from jax import lax
from jax.experimental import pallas as pl
from jax.experimental.pallas import tpu as pltpu

N_DEV = 16
EPS = 1e-5


def kernel(x, t_emb, W_scale, W_shift):
    b, s, c = x.shape
    c_global = c * N_DEV

    def body(x_ref, t_ref, ws_ref, wsh_ref, out_ref,
             statbuf, send_sems, recv_sems):
        my = lax.axis_index("i")

        barrier = pltpu.get_barrier_semaphore()
        for d in range(1, N_DEV):
            peer = lax.rem(my + d, N_DEV)
            pl.semaphore_signal(
                barrier, inc=1,
                device_id=(peer,), device_id_type=pl.DeviceIdType.MESH,
            )

        xv = x_ref[...]
        s1 = jnp.sum(xv, axis=-1)
        s2 = jnp.sum(xv * xv, axis=-1)
        statbuf[pl.ds(my, 1)] = jnp.stack([s1, s2], axis=0)[None]

        scale = jnp.dot(t_ref[...], ws_ref[...],
                        preferred_element_type=jnp.float32)
        shift = jnp.dot(t_ref[...], wsh_ref[...],
                        preferred_element_type=jnp.float32)

        pl.semaphore_wait(barrier, N_DEV - 1)

        sends = []
        for d in range(1, N_DEV):
            dst = lax.rem(my + d, N_DEV)
            rdma = pltpu.make_async_remote_copy(
                src_ref=statbuf.at[my],
                dst_ref=statbuf.at[my],
                send_sem=send_sems.at[d - 1],
                recv_sem=recv_sems.at[my],
                device_id=(dst,),
                device_id_type=pl.DeviceIdType.MESH,
            )
            rdma.start()
            sends.append(rdma)

        for d in range(1, N_DEV):
            src = lax.rem(my + d, N_DEV)
            recv = pltpu.make_async_remote_copy(
                src_ref=statbuf.at[my],
                dst_ref=statbuf.at[src],
                send_sem=send_sems.at[d - 1],
                recv_sem=recv_sems.at[src],
                device_id=(my,),
                device_id_type=pl.DeviceIdType.MESH,
            )
            recv.wait_recv()
        for rdma in sends:
            rdma.wait_send()

        tot = jnp.sum(statbuf[...], axis=0)
        mean = tot[0] / c_global
        var = tot[1] / c_global - mean * mean
        inv = lax.rsqrt(var + EPS)
        h = (xv - mean[..., None]) * inv[..., None]
        out_ref[...] = h * (1.0 + scale[:, None, :]) + shift[:, None, :]

    return pl.pallas_call(
        body,
        out_shape=jax.ShapeDtypeStruct((b, s, c), jnp.float32),
        in_specs=[
            pl.BlockSpec(memory_space=pltpu.VMEM),
            pl.BlockSpec(memory_space=pltpu.VMEM),
            pl.BlockSpec(memory_space=pltpu.VMEM),
            pl.BlockSpec(memory_space=pltpu.VMEM),
        ],
        out_specs=pl.BlockSpec(memory_space=pltpu.VMEM),
        scratch_shapes=[
            pltpu.VMEM((N_DEV, 2, b, s), jnp.float32),
            pltpu.SemaphoreType.DMA((N_DEV - 1,)),
            pltpu.SemaphoreType.DMA((N_DEV,)),
        ],
        compiler_params=pltpu.CompilerParams(collective_id=0),
    )(x, t_emb, W_scale, W_shift)
